# Initial kernel scaffold; baseline (speedup 1.0000x reference)
import jax
import jax.numpy as jnp
from jax.experimental import pallas as pl


def kernel(z, codebook):
    raise NotImplementedError("write your pallas kernel here")


# speedup vs baseline: 8.3196x; 8.3196x over previous
import functools

import jax
import jax.numpy as jnp
from jax.experimental import pallas as pl

NUM_CODES = 1024
NUM_LEVELS = 4
LATENT_DIM = 32
BLOCK_B = 1024
LO = 128
HI = NUM_CODES // LO


N_CHUNK = 2


def _rvq_body(z_ref, cb_ref, cbg_ref, qst_ref, idx_ref, q_ref):
    cw = BLOCK_B // N_CHUNK
    zs = [z_ref[pl.ds(i * cw, cw), :] for i in range(N_CHUNK)]
    ones = jnp.ones((cw, 1), jnp.float32)
    rs = list(zs)
    qs = [jnp.zeros_like(z) for z in zs]
    idxs = [[] for _ in range(N_CHUNK)]
    for level in range(NUM_LEVELS):
        c = cb_ref[level]
        cn = jnp.sum(c * c, axis=1, keepdims=True)
        cmat = jnp.concatenate([-2.0 * c, cn], axis=1)
        for i in range(N_CHUNK):
            ra = jnp.concatenate([rs[i], ones], axis=1)
            scores = jax.lax.dot_general(
                ra, cmat, (((1,), (1,)), ((), ())),
                preferred_element_type=jnp.float32,
                precision=jax.lax.Precision.HIGHEST)
            idx = jnp.argmin(scores, axis=1).astype(jnp.int32)
            idxs[i].append(idx)
            lo = (idx & (LO - 1))[:, None]
            hi = (idx >> 7)[:, None]
            onehot_lo = (jax.lax.broadcasted_iota(
                jnp.int32, (cw, LO), 1) == lo).astype(jnp.float32)
            cand = jax.lax.dot_general(
                onehot_lo, cbg_ref[level], (((1,), (0,)), ((), ())),
                preferred_element_type=jnp.float32,
                precision=jax.lax.Precision.HIGHEST)
            lq = jnp.zeros_like(zs[i])
            for h in range(HI):
                lq = lq + jnp.where(hi == h,
                                    cand[:, h * LATENT_DIM:(h + 1) * LATENT_DIM],
                                    0.0)
            qs[i] = qs[i] + lq
            rs[i] = rs[i] - lq
    for i in range(N_CHUNK):
        sl = pl.ds(i * cw, cw)
        q_ref[sl, :] = qs[i]
        qst_ref[sl, :] = zs[i] + (qs[i] - zs[i])
        idx_ref[sl, :] = jnp.concatenate([x[:, None] for x in idxs[i]], axis=1)


@functools.partial(jax.jit, static_argnames=())
def kernel(z, codebook):
    batch, d = z.shape
    cbg = codebook.reshape(NUM_LEVELS, HI, LO, d).transpose(0, 2, 1, 3)
    cbg = cbg.reshape(NUM_LEVELS, LO, HI * d)
    grid = (batch // BLOCK_B,)
    qst, idx, q = pl.pallas_call(
        _rvq_body,
        grid=grid,
        in_specs=[
            pl.BlockSpec((BLOCK_B, d), lambda i: (i, 0)),
            pl.BlockSpec((NUM_LEVELS, NUM_CODES, d), lambda i: (0, 0, 0)),
            pl.BlockSpec((NUM_LEVELS, LO, HI * d), lambda i: (0, 0, 0)),
        ],
        out_specs=[
            pl.BlockSpec((BLOCK_B, d), lambda i: (i, 0)),
            pl.BlockSpec((BLOCK_B, NUM_LEVELS), lambda i: (i, 0)),
            pl.BlockSpec((BLOCK_B, d), lambda i: (i, 0)),
        ],
        out_shape=[
            jax.ShapeDtypeStruct((batch, d), jnp.float32),
            jax.ShapeDtypeStruct((batch, NUM_LEVELS), jnp.int32),
            jax.ShapeDtypeStruct((batch, d), jnp.float32),
        ],
    )(z, codebook, cbg)
    return (qst, idx, q)
